# baseline (speedup 1.0000x reference)
import jax
import jax.numpy as jnp
from jax.experimental import pallas as pl
from jax.experimental.pallas import tpu as pltpu

_CH = 200
_NBUF = 4
_AUTO_BLOCKS = 26
_LAG = 2


def _fused_kernel(adj_blk_ref, adj_hbm_ref, feature_ref, w_ref, b_ref, out_ref,
                  buf_ref, support_ref, sems):
    i = pl.program_id(0)
    n = out_ref.shape[0]
    base = _AUTO_BLOCKS * _CH
    nman = (n - base) // _CH

    @pl.when(i == 0)
    def _():
        for s in range(_NBUF):
            pltpu.make_async_copy(
                adj_hbm_ref.at[pl.ds(base + s * _CH, _CH), :],
                buf_ref.at[s],
                sems.at[s],
            ).start()
        support_ref[...] = jnp.maximum(
            jnp.dot(feature_ref[...], w_ref[...], preferred_element_type=jnp.float32),
            0.0,
        )

    b_row = b_ref[...]

    acc0 = jnp.dot(adj_blk_ref[...], support_ref[...], preferred_element_type=jnp.float32)
    out_ref[pl.ds(i * _CH, _CH), :] = (
        acc0 + support_ref[pl.ds(i * _CH, _CH), :] + b_row
    )

    @pl.when(i >= _LAG)
    def _():
        j = i - _LAG
        slot = jax.lax.rem(j, _NBUF)
        pltpu.make_async_copy(
            adj_hbm_ref.at[pl.ds(base + j * _CH, _CH), :], buf_ref.at[slot], sems.at[slot]
        ).wait()
        acc1 = jnp.dot(buf_ref[slot], support_ref[...], preferred_element_type=jnp.float32)
        r0 = base + j * _CH
        out_ref[pl.ds(r0, _CH), :] = (
            acc1 + support_ref[pl.ds(r0, _CH), :] + b_row
        )

        @pl.when(j + _NBUF < nman)
        def _():
            nxt = j + _NBUF
            pltpu.make_async_copy(
                adj_hbm_ref.at[pl.ds(base + nxt * _CH, _CH), :],
                buf_ref.at[slot],
                sems.at[slot],
            ).start()


def kernel(feature, adj, W, b):
    n, f_in = feature.shape
    d = W.shape[1]
    b2 = b.reshape(1, d)
    grid = (_AUTO_BLOCKS,)

    out = pl.pallas_call(
        _fused_kernel,
        grid=grid,
        in_specs=[
            pl.BlockSpec((_CH, n), lambda i: (i, 0)),
            pl.BlockSpec(memory_space=pltpu.HBM),
            pl.BlockSpec((n, f_in), lambda i: (0, 0)),
            pl.BlockSpec((f_in, d), lambda i: (0, 0)),
            pl.BlockSpec((1, d), lambda i: (0, 0)),
        ],
        out_specs=pl.BlockSpec(memory_space=pltpu.VMEM),
        out_shape=jax.ShapeDtypeStruct((n, d), jnp.float32),
        scratch_shapes=[
            pltpu.VMEM((_NBUF, _CH, n), jnp.float32),
            pltpu.VMEM((n, d), jnp.float32),
            pltpu.SemaphoreType.DMA((_NBUF,)),
        ],
        compiler_params=pltpu.CompilerParams(
            dimension_semantics=("arbitrary",),
            vmem_limit_bytes=100 * 1024 * 1024,
            skip_device_barrier=True,
        ),
    )(adj, adj, feature, W, b2)

    return out


# speedup vs baseline: 1.0463x; 1.0463x over previous
import jax
import jax.numpy as jnp
from jax.experimental import pallas as pl
from jax.experimental.pallas import tpu as pltpu


def _fused_kernel(adj_ref, feature_ref, w_ref, b_ref, out_ref, support_ref):
    i = pl.program_id(0)

    @pl.when(i == 0)
    def _():
        support_ref[...] = jnp.maximum(
            jnp.dot(feature_ref[...], w_ref[...], preferred_element_type=jnp.float32),
            0.0,
        )

    br = out_ref.shape[0]
    acc = jnp.dot(adj_ref[...], support_ref[...], preferred_element_type=jnp.float32)
    out_ref[...] = acc + support_ref[pl.ds(i * br, br), :] + b_ref[...]


def kernel(feature, adj, W, b):
    n, f_in = feature.shape
    d = W.shape[1]
    b2 = b.reshape(1, d)

    br = 400
    grid = (n // br,)

    out = pl.pallas_call(
        _fused_kernel,
        grid=grid,
        in_specs=[
            pl.BlockSpec((br, n), lambda i: (i, 0)),
            pl.BlockSpec((n, f_in), lambda i: (0, 0)),
            pl.BlockSpec((f_in, d), lambda i: (0, 0)),
            pl.BlockSpec((1, d), lambda i: (0, 0)),
        ],
        out_specs=pl.BlockSpec((br, d), lambda i: (i, 0)),
        out_shape=jax.ShapeDtypeStruct((n, d), jnp.float32),
        scratch_shapes=[
            pltpu.VMEM((n, d), jnp.float32),
        ],
        compiler_params=pltpu.CompilerParams(
            dimension_semantics=("arbitrary",),
            skip_device_barrier=True,
        ),
    )(adj, feature, W, b2)

    return out
